# baseline (speedup 1.0000x reference)
import functools

import jax
import jax.numpy as jnp
from jax import lax
from jax.experimental import pallas as pl
from jax.experimental.pallas import tpu as pltpu
from jax.experimental.pallas import tpu_sc as plsc

M_COLS = 28
MPAD = 32
ALPHA_SLOPE = 0.2
BN = 1000
PAR = 2
NSUB = 32
ROWS_PER = 313


def _stage_a(x_ref, x28_ref, w_ref, a_ref, s1_ref, h28_ref, s2_ref, w1_sc):
    j = pl.program_id(1)
    d = w_ref.shape[0]

    @pl.when(j == 0)
    def _prologue():
        w = w_ref[...]
        h28 = jax.lax.dot_general(x28_ref[...], w, (((1,), (1,)), ((), ())),
                                  preferred_element_type=jnp.float32)
        h28_ref[...] = h28
        w1_sc[...] = jnp.dot(a_ref[:, :d], w, preferred_element_type=jnp.float32)
        s2_ref[...] = jax.lax.dot_general(a_ref[:, d:], h28,
                                          (((1,), (1,)), ((), ())),
                                          preferred_element_type=jnp.float32)

    s1 = jnp.sum(x_ref[...] * w1_sc[...], axis=1, keepdims=True)
    s1_ref[...] = jnp.broadcast_to(s1, (s1.shape[0], MPAD))


def _stage_b(s1_hbm, s2_hbm, edge_hbm, e_hbm, s1_v, s2_v, edge_v, e_v):
    wid = lax.axis_index("s") * 2 + lax.axis_index("c")
    start = wid * (ROWS_PER * MPAD)
    pltpu.sync_copy(s1_hbm.at[pl.ds(start, ROWS_PER * MPAD)], s1_v)
    pltpu.sync_copy(s2_hbm, s2_v)
    pltpu.sync_copy(edge_hbm.at[pl.ds(start, ROWS_PER * MPAD)], edge_v)
    s2lo = s2_v[pl.ds(0, 16)]
    s2hi = s2_v[pl.ds(16, 16)]

    def body(i, carry):
        for half, s2h in ((0, s2lo), (1, s2hi)):
            off = i * MPAD + half * 16
            lg = s1_v[pl.ds(off, 16)] + s2h
            lr = jnp.where(lg >= 0, lg, ALPHA_SLOPE * lg)
            ev = jnp.where(edge_v[pl.ds(off, 16)] != 0, jnp.exp(-lr), 0.0)
            e_v[pl.ds(off, 16)] = ev
        return carry

    lax.fori_loop(0, ROWS_PER, body, 0)
    pltpu.sync_copy(e_v, e_hbm.at[pl.ds(start, ROWS_PER * MPAD)])


def _stage_c(e_ref, h28_ref, out_ref):
    e = e_ref[...]
    rowsum = jnp.sum(e, axis=1, keepdims=True)
    hp = jnp.dot(e, h28_ref[...], preferred_element_type=jnp.float32) / rowsum
    out_ref[...] = jnp.where(hp > 0, hp, jnp.exp(hp) - 1.0)


def kernel(x, edge, W, a):
    n, d_in = x.shape
    d_out = W.shape[0]
    npad = NSUB * ROWS_PER
    steps = n // (BN * PAR)

    s1b, h28, _s2 = pl.pallas_call(
        _stage_a,
        grid=(PAR, steps),
        in_specs=[
            pl.BlockSpec((BN, d_in), lambda i, j: (i * (n // (BN * PAR)) + j, 0)),
            pl.BlockSpec((MPAD, d_in), lambda i, j: (0, 0)),
            pl.BlockSpec((d_out, d_in), lambda i, j: (0, 0)),
            pl.BlockSpec((1, 2 * d_out), lambda i, j: (0, 0)),
        ],
        out_specs=[
            pl.BlockSpec((BN, MPAD), lambda i, j: (i * (n // (BN * PAR)) + j, 0)),
            pl.BlockSpec((MPAD, d_out), lambda i, j: (0, 0)),
            pl.BlockSpec((1, MPAD), lambda i, j: (0, 0)),
        ],
        out_shape=[
            jax.ShapeDtypeStruct((npad, MPAD), jnp.float32),
            jax.ShapeDtypeStruct((MPAD, d_out), jnp.float32),
            jax.ShapeDtypeStruct((1, MPAD), jnp.float32),
        ],
        scratch_shapes=[pltpu.VMEM((1, d_in), jnp.float32)],
        compiler_params=pltpu.CompilerParams(
            dimension_semantics=("parallel", "arbitrary")),
    )(x, x, W, a)

    s1_flat = s1b.reshape(-1)
    edge_flat = jnp.pad(edge, ((0, npad - n), (0, MPAD - M_COLS))).reshape(-1)
    s2_flat = _s2.reshape(-1)

    sc_edge = functools.partial(
        pl.kernel,
        mesh=plsc.VectorSubcoreMesh(core_axis_name="c", subcore_axis_name="s"),
        out_type=jax.ShapeDtypeStruct((npad * MPAD,), jnp.float32),
        scratch_types=[
            pltpu.VMEM((ROWS_PER * MPAD,), jnp.float32),
            pltpu.VMEM((MPAD,), jnp.float32),
            pltpu.VMEM((ROWS_PER * MPAD,), jnp.int32),
            pltpu.VMEM((ROWS_PER * MPAD,), jnp.float32),
        ],
    )(_stage_b)
    e_flat = sc_edge(s1_flat, s2_flat, edge_flat)
    e_mat = e_flat.reshape(npad, MPAD)

    return pl.pallas_call(
        _stage_c,
        grid=(PAR, steps),
        in_specs=[
            pl.BlockSpec((BN, MPAD), lambda i, j: (i * (n // (BN * PAR)) + j, 0)),
            pl.BlockSpec((MPAD, d_out), lambda i, j: (0, 0)),
        ],
        out_specs=pl.BlockSpec((BN, d_out), lambda i, j: (i * (n // (BN * PAR)) + j, 0)),
        out_shape=jax.ShapeDtypeStruct((n, d_out), jnp.float32),
        compiler_params=pltpu.CompilerParams(
            dimension_semantics=("parallel", "arbitrary")),
    )(e_mat, h28)


# speedup vs baseline: 2.7856x; 2.7856x over previous
import jax
import jax.numpy as jnp
from jax.experimental import pallas as pl
from jax.experimental.pallas import tpu as pltpu

M_COLS = 28
MPAD = 32
ALPHA_SLOPE = 0.2
BN = 2000
PAR = 1


def _gat_kernel(x_ref, edge_ref, x28_ref, w_ref, a_ref,
                out_ref, h28_sc, w1_sc, s2_sc):
    j = pl.program_id(1)
    d = w_ref.shape[0]

    @pl.when(j == 0)
    def _prologue():
        w = w_ref[...]
        h28 = jax.lax.dot_general(x28_ref[...], w, (((1,), (1,)), ((), ())),
                                  preferred_element_type=jnp.float32)
        h28_sc[...] = h28
        w1_sc[...] = jnp.dot(a_ref[:, :d], w, preferred_element_type=jnp.float32)
        s2_sc[...] = jax.lax.dot_general(a_ref[:, d:], h28,
                                         (((1,), (1,)), ((), ())),
                                         preferred_element_type=jnp.float32)

    s1 = jnp.sum(x_ref[...] * w1_sc[...], axis=1, keepdims=True)
    logits = s1 + s2_sc[...]
    lrelu = jnp.where(logits >= 0, logits, ALPHA_SLOPE * logits)
    edge_p = jnp.concatenate(
        [edge_ref[...], jnp.zeros((edge_ref.shape[0], MPAD - M_COLS),
                                  jnp.int32)], axis=1)
    e = jnp.where(edge_p != 0, jnp.exp(-lrelu), 0.0)
    rowsum = jnp.sum(e, axis=1, keepdims=True)
    hp = jnp.dot(e, h28_sc[...], preferred_element_type=jnp.float32) / rowsum
    out_ref[...] = jnp.where(hp > 0, hp, jnp.exp(hp) - 1.0)


def kernel(x, edge, W, a):
    n, d_in = x.shape
    d_out = W.shape[0]
    steps = n // (BN * PAR)
    return pl.pallas_call(
        _gat_kernel,
        grid=(PAR, steps),
        in_specs=[
            pl.BlockSpec((BN, d_in), lambda i, j: (i * (n // (BN * PAR)) + j, 0)),
            pl.BlockSpec((BN, M_COLS), lambda i, j: (i * (n // (BN * PAR)) + j, 0)),
            pl.BlockSpec((MPAD, d_in), lambda i, j: (0, 0)),
            pl.BlockSpec((d_out, d_in), lambda i, j: (0, 0)),
            pl.BlockSpec((1, 2 * d_out), lambda i, j: (0, 0)),
        ],
        out_specs=pl.BlockSpec((BN, d_out), lambda i, j: (i * (n // (BN * PAR)) + j, 0)),
        out_shape=jax.ShapeDtypeStruct((n, d_out), jnp.float32),
        scratch_shapes=[
            pltpu.VMEM((MPAD, d_out), jnp.float32),
            pltpu.VMEM((1, d_out), jnp.float32),
            pltpu.VMEM((1, MPAD), jnp.float32),
        ],
        compiler_params=pltpu.CompilerParams(
            dimension_semantics=("parallel", "arbitrary")),
    )(x, edge, x, W, a)


# speedup vs baseline: 3.0099x; 1.0805x over previous
import jax
import jax.numpy as jnp
from jax.experimental import pallas as pl
from jax.experimental.pallas import tpu as pltpu

M_COLS = 28
MPAD = 32
ALPHA_SLOPE = 0.2
BN = 5000
PAR = 1


def _gat_kernel(x_ref, edge_ref, x28_ref, w_ref, a_ref,
                out_ref, h28_sc, w1_sc, s2_sc):
    j = pl.program_id(1)
    d = w_ref.shape[0]

    @pl.when(j == 0)
    def _prologue():
        w = w_ref[...]
        h28 = jax.lax.dot_general(x28_ref[...], w, (((1,), (1,)), ((), ())),
                                  preferred_element_type=jnp.float32)
        h28_sc[...] = h28
        w1_sc[...] = jnp.dot(a_ref[:, :d], w, preferred_element_type=jnp.float32)
        s2_sc[...] = jax.lax.dot_general(a_ref[:, d:], h28,
                                         (((1,), (1,)), ((), ())),
                                         preferred_element_type=jnp.float32)

    s1 = jnp.sum(x_ref[...] * w1_sc[...], axis=1, keepdims=True)
    logits = s1 + s2_sc[...]
    lrelu = jnp.where(logits >= 0, logits, ALPHA_SLOPE * logits)
    edge_p = jnp.concatenate(
        [edge_ref[...], jnp.zeros((edge_ref.shape[0], MPAD - M_COLS),
                                  jnp.int32)], axis=1)
    e = jnp.where(edge_p != 0, jnp.exp(-lrelu), 0.0)
    rowsum = jnp.sum(e, axis=1, keepdims=True)
    hp = jnp.dot(e, h28_sc[...], preferred_element_type=jnp.float32) / rowsum
    out_ref[...] = jnp.where(hp > 0, hp, jnp.exp(hp) - 1.0)


def kernel(x, edge, W, a):
    n, d_in = x.shape
    d_out = W.shape[0]
    steps = n // (BN * PAR)
    return pl.pallas_call(
        _gat_kernel,
        grid=(PAR, steps),
        in_specs=[
            pl.BlockSpec((BN, d_in), lambda i, j: (i * (n // (BN * PAR)) + j, 0)),
            pl.BlockSpec((BN, M_COLS), lambda i, j: (i * (n // (BN * PAR)) + j, 0)),
            pl.BlockSpec((MPAD, d_in), lambda i, j: (0, 0)),
            pl.BlockSpec((d_out, d_in), lambda i, j: (0, 0)),
            pl.BlockSpec((1, 2 * d_out), lambda i, j: (0, 0)),
        ],
        out_specs=pl.BlockSpec((BN, d_out), lambda i, j: (i * (n // (BN * PAR)) + j, 0)),
        out_shape=jax.ShapeDtypeStruct((n, d_out), jnp.float32),
        scratch_shapes=[
            pltpu.VMEM((MPAD, d_out), jnp.float32),
            pltpu.VMEM((1, d_out), jnp.float32),
            pltpu.VMEM((1, MPAD), jnp.float32),
        ],
        compiler_params=pltpu.CompilerParams(
            dimension_semantics=("parallel", "arbitrary")),
    )(x, edge, x, W, a)
